# Initial kernel scaffold; baseline (speedup 1.0000x reference)
import jax
import jax.numpy as jnp
from jax.experimental import pallas as pl


def kernel(inputs, codebook, training):
    raise NotImplementedError("write your pallas kernel here")


# speedup vs baseline: 6.7367x; 6.7367x over previous
import functools

import jax
import jax.numpy as jnp
from jax import lax
from jax.experimental import pallas as pl
from jax.experimental.pallas import tpu as pltpu

K = 512
D = 64


def _vq_tc_body(x_ref, cbt_ref, cb_ref, out_ref):
    x = x_ref[...]
    cbt = cbt_ref[...]
    cb = cb_ref[...]
    xc = lax.dot_general(
        x, cbt, (((1,), (0,)), ((), ())),
        preferred_element_type=jnp.float32,
        precision=lax.Precision.HIGHEST,
    )
    cnorm = jnp.sum(cbt * cbt, axis=0)[None, :]
    scores = cnorm - 2.0 * xc
    m = jnp.min(scores, axis=1, keepdims=True)
    iota = lax.broadcasted_iota(jnp.int32, scores.shape, 1)
    tk = jnp.min(jnp.where(scores == m, iota, K), axis=1, keepdims=True)
    onehot = (iota == tk).astype(jnp.float32)
    emb = lax.dot_general(
        onehot, cb, (((1,), (0,)), ((), ())),
        preferred_element_type=jnp.float32,
        precision=lax.Precision.HIGHEST,
    )
    out_ref[...] = x + (emb - x)


@functools.partial(jax.jit, static_argnames=("interpret",))
def _vq_tc(x2d, cbt, cb, interpret=False):
    n = x2d.shape[0]
    return pl.pallas_call(
        _vq_tc_body,
        out_shape=jax.ShapeDtypeStruct((n, D), jnp.float32),
        interpret=interpret,
    )(x2d, cbt, cb)


def kernel(inputs, codebook, training):
    x2d = inputs.reshape(-1, D)
    out = _vq_tc(x2d, codebook.T, codebook)
    return out.reshape(inputs.shape)
